# Initial kernel scaffold; baseline (speedup 1.0000x reference)
import jax
import jax.numpy as jnp
from jax.experimental import pallas as pl


def kernel(p, x, W_enc0, W_enc1, W_enc2, W_dec1, W_dec0, W_fusion, W_edge, W_ee, W_cls1, b_cls1, gamma, beta, W_cls2, b_cls2, o):
    raise NotImplementedError("write your pallas kernel here")


# speedup vs baseline: 5.6558x; 5.6558x over previous
import functools

import jax
import jax.numpy as jnp
from jax import lax
from jax.experimental import pallas as pl
from jax.experimental.pallas import tpu as pltpu
from jax.experimental.pallas import tpu_sc as plsc

_BIG = 3.0e38


def _knn_body(k, with_w, nr, q_ref, r_ref, idx_ref, *maybe_w_ref):
    q = q_ref[...]
    r = r_ref[...]
    qn = jnp.sum(q * q, axis=1, keepdims=True)
    rn = jnp.sum(r * r, axis=1)[None, :]
    qr = lax.dot_general(q, r, (((1,), (1,)), ((), ())),
                         preferred_element_type=jnp.float32)
    d = qn - 2.0 * qr + rn
    iota = lax.broadcasted_iota(jnp.int32, d.shape, 1)
    cols = []
    vals = []
    for _ in range(k):
        m = jnp.min(d, axis=1, keepdims=True)
        cand = jnp.where(d == m, iota, nr)
        col = jnp.min(cand, axis=1, keepdims=True)
        cols.append(col)
        vals.append(m)
        d = jnp.where(iota == col, _BIG, d)
    idx_ref[...] = jnp.concatenate(cols, axis=1)
    if with_w:
        d2 = jnp.maximum(jnp.concatenate(vals, axis=1), 0.0)
        w = 1.0 / (d2 + 1e-8)
        maybe_w_ref[0][...] = w / jnp.sum(w, axis=1, keepdims=True)


def _knn(q, r, k, with_w, block_rows=128):
    nq = q.shape[0]
    nr = r.shape[0]
    bq = min(block_rows, nq)
    grid = (nq // bq,)
    out_shape = [jax.ShapeDtypeStruct((nq, k), jnp.int32)]
    out_specs = [pl.BlockSpec((bq, k), lambda i: (i, 0))]
    if with_w:
        out_shape.append(jax.ShapeDtypeStruct((nq, k), jnp.float32))
        out_specs.append(pl.BlockSpec((bq, k), lambda i: (i, 0)))
    res = pl.pallas_call(
        functools.partial(_knn_body, k, with_w, nr),
        grid=grid,
        in_specs=[pl.BlockSpec((bq, 3), lambda i: (i, 0)),
                  pl.BlockSpec((nr, 3), lambda i: (0, 0))],
        out_specs=out_specs,
        out_shape=out_shape,
    )(q, r)
    return res if with_w else res[0]


def _dense_body(n_in, *refs):
    out_ref = refs[-1]
    acc = None
    for i in range(n_in):
        part = jnp.dot(refs[i][...], refs[n_in + i][...],
                       preferred_element_type=jnp.float32)
        acc = part if acc is None else acc + part
    out_ref[...] = jnp.maximum(acc, 0.0)


def _dense_relu(xs, ws, block_rows=512):
    m = xs[0].shape[0]
    bm = min(block_rows, m)
    n = ws[0].shape[1]
    in_specs = []
    for x in xs:
        kd = x.shape[1]
        in_specs.append(pl.BlockSpec((bm, kd), lambda i: (i, 0)))
    for w in ws:
        in_specs.append(pl.BlockSpec(w.shape, lambda i: (0, 0)))
    return pl.pallas_call(
        functools.partial(_dense_body, len(xs)),
        grid=(m // bm,),
        in_specs=in_specs,
        out_specs=pl.BlockSpec((bm, n), lambda i: (i, 0)),
        out_shape=jax.ShapeDtypeStruct((m, n), jnp.float32),
    )(*xs, *ws)


def _edge_body(a_ref, b_ref, w_ref, o_ref):
    o_ref[...] = jnp.maximum(
        jnp.dot(a_ref[...] - b_ref[...], w_ref[...],
                preferred_element_type=jnp.float32), 0.0)


def _edge_mm(a, b, w, block_rows=512):
    m, kd = a.shape
    n = w.shape[1]
    bm = min(block_rows, m)
    return pl.pallas_call(
        _edge_body,
        grid=(m // bm,),
        in_specs=[pl.BlockSpec((bm, kd), lambda i: (i, 0)),
                  pl.BlockSpec((bm, kd), lambda i: (i, 0)),
                  pl.BlockSpec(w.shape, lambda i: (0, 0))],
        out_specs=pl.BlockSpec((bm, n), lambda i: (i, 0)),
        out_shape=jax.ShapeDtypeStruct((m, n), jnp.float32),
    )(a, b, w)


def _cls_body(x_ref, w1_ref, b1_ref, g_ref, be_ref, w2_ref, b2_ref, o_ref):
    h = jnp.dot(x_ref[...], w1_ref[...],
                preferred_element_type=jnp.float32) + b1_ref[...]
    mu = jnp.mean(h, axis=0, keepdims=True)
    var = jnp.mean((h - mu) * (h - mu), axis=0, keepdims=True)
    h = g_ref[...] * (h - mu) / jnp.sqrt(var + 1e-5) + be_ref[...]
    h = jnp.maximum(h, 0.0)
    o_ref[...] = jnp.dot(h, w2_ref[...],
                         preferred_element_type=jnp.float32) + b2_ref[...]


def _classifier(x, w1, b1, g, be, w2, b2):
    m, d = x.shape
    n = w2.shape[1]
    full = lambda s: pl.BlockSpec(s, lambda: tuple(0 for _ in s))
    return pl.pallas_call(
        _cls_body,
        in_specs=[full((m, d)), full(w1.shape), full((1, d)), full((1, d)),
                  full((1, d)), full(w2.shape), full((1, n))],
        out_specs=full((m, n)),
        out_shape=jax.ShapeDtypeStruct((m, n), jnp.float32),
    )(x, w1, b1.reshape(1, -1), g.reshape(1, -1), be.reshape(1, -1),
      w2, b2.reshape(1, -1))


def _pad128(table):
    d = table.shape[1]
    dp = (d + 127) // 128 * 128
    if dp != d:
        table = jnp.pad(table, ((0, 0), (0, dp - d)))
    return table, d


def _sc_poolmax(table, idx):
    table, d_orig = _pad128(table)
    q_tot, kk = idx.shape
    _, d = table.shape
    info = plsc.get_sparse_core_info()
    nw = info.num_cores * info.num_subcores
    qpw = q_tot // nw
    b = max(1, min(128 // kk, qpw))
    nqb = qpw // b
    mesh = plsc.VectorSubcoreMesh(core_axis_name="c", subcore_axis_name="s")

    @functools.partial(
        pl.kernel, mesh=mesh,
        out_type=jax.ShapeDtypeStruct((q_tot, d), jnp.float32),
        scratch_types=[
            pltpu.VMEM((qpw * kk,), jnp.int32),
            pltpu.VMEM((b * kk, d), jnp.float32),
            pltpu.VMEM((b, d), jnp.float32),
            pltpu.SemaphoreType.DMA,
        ],
    )
    def kern(table_hbm, idx_hbm, out_hbm, idx_v, rows_v, out_v, sem):
        wid = lax.axis_index("s") * info.num_cores + lax.axis_index("c")
        base = wid * qpw
        pltpu.sync_copy(idx_hbm.at[pl.ds(base * kk, qpw * kk)], idx_v)

        def qb_body(qb):
            pltpu.async_copy(
                table_hbm.at[idx_v.at[pl.ds(qb * b * kk, b * kk)]],
                rows_v, sem).wait()

            def b_body(bi):
                for dch in range(d // 16):
                    sl = pl.ds(dch * 16, 16)
                    acc = rows_v[bi * kk, sl]
                    for j in range(1, kk):
                        acc = jnp.maximum(acc, rows_v[bi * kk + j, sl])
                    out_v[bi, sl] = acc
            lax.fori_loop(0, b, lambda i, _: (b_body(i), 0)[1], 0)
            pltpu.sync_copy(out_v, out_hbm.at[pl.ds(base + qb * b, b)])

        lax.fori_loop(0, nqb, lambda i, _: (qb_body(i), 0)[1], 0)

    out = kern(table, idx.reshape(-1))
    return out[:, :d_orig] if d_orig != d else out


def _sc_interp(table, idx, w):
    table, d_orig = _pad128(table)
    q_tot, kk = idx.shape
    _, d = table.shape
    info = plsc.get_sparse_core_info()
    nw = info.num_cores * info.num_subcores
    qpw = q_tot // nw
    b = max(1, min(128 // kk, qpw))
    nqb = qpw // b
    mesh = plsc.VectorSubcoreMesh(core_axis_name="c", subcore_axis_name="s")

    @functools.partial(
        pl.kernel, mesh=mesh,
        out_type=jax.ShapeDtypeStruct((q_tot, d), jnp.float32),
        scratch_types=[
            pltpu.VMEM((qpw * kk,), jnp.int32),
            pltpu.VMEM((qpw * kk + 16,), jnp.float32),
            pltpu.VMEM((b * kk, d), jnp.float32),
            pltpu.VMEM((b, d), jnp.float32),
            pltpu.SemaphoreType.DMA,
        ],
    )
    def kern(table_hbm, idx_hbm, w_hbm, out_hbm, idx_v, w_v, rows_v, out_v,
             sem):
        wid = lax.axis_index("s") * info.num_cores + lax.axis_index("c")
        base = wid * qpw
        pltpu.sync_copy(idx_hbm.at[pl.ds(base * kk, qpw * kk)], idx_v)
        pltpu.sync_copy(w_hbm.at[pl.ds(base * kk, qpw * kk)],
                        w_v.at[pl.ds(0, qpw * kk)])

        def qb_body(qb):
            pltpu.async_copy(
                table_hbm.at[idx_v.at[pl.ds(qb * b * kk, b * kk)]],
                rows_v, sem).wait()

            def b_body(bi):
                qi = qb * b + bi
                wvec = w_v[pl.ds(qi * kk, 16)]
                for j in range(kk):
                    wj = wvec[j]
                    for dch in range(d // 16):
                        sl = pl.ds(dch * 16, 16)
                        row = rows_v[bi * kk + j, sl]
                        if j == 0:
                            out_v[bi, sl] = wj * row
                        else:
                            out_v[bi, sl] = out_v[bi, sl] + wj * row
            lax.fori_loop(0, b, lambda i, _: (b_body(i), 0)[1], 0)
            pltpu.sync_copy(out_v, out_hbm.at[pl.ds(base + qb * b, b)])

        lax.fori_loop(0, nqb, lambda i, _: (qb_body(i), 0)[1], 0)

    out = kern(table, idx.reshape(-1), w.reshape(-1))
    return out[:, :d_orig] if d_orig != d else out


def kernel(p, x, W_enc0, W_enc1, W_enc2, W_dec1, W_dec0, W_fusion, W_edge,
           W_ee, W_cls1, b_cls1, gamma, beta, W_cls2, b_cls2, o):
    del o
    n0 = p.shape[0]
    d0_dim, d1_dim, d2_dim = W_enc0.shape[1], W_enc1.shape[1], W_enc2.shape[1]

    x0 = _dense_relu([x], [W_enc0])
    p1 = p[::4]
    idx1 = _knn(p1, p, 16, with_w=False)
    x1 = _dense_relu([_sc_poolmax(x0, idx1)], [W_enc1])
    p2 = p1[::4]
    idx2 = _knn(p2, p1, 16, with_w=False)
    x2 = _dense_relu([_sc_poolmax(x1, idx2)], [W_enc2])

    idx_u1, w_u1 = _knn(p1, p2, 8, with_w=True)
    up1 = _sc_interp(x2, idx_u1, w_u1)
    d1 = _dense_relu([up1, x1],
                     [W_dec1[:d2_dim], W_dec1[d2_dim:]])
    idx_u0, w_u0 = _knn(p, p1, 8, with_w=True)
    up0 = _sc_interp(d1, idx_u0, w_u0)
    d0 = _dense_relu([up0, x0],
                     [W_dec0[:d1_dim], W_dec0[d1_dim:]])

    ms1 = _sc_interp(x1, idx_u0, w_u0)
    idx_m2, w_m2 = _knn(p, p2, 8, with_w=True)
    ms2 = _sc_interp(x2, idx_m2, w_m2)
    ws = [W_fusion[:d0_dim], W_fusion[d0_dim:d0_dim + d1_dim],
          W_fusion[d0_dim + d1_dim:]]
    ms_new = _dense_relu([x0, ms1, ms2], ws)

    idx_e = _knn(p, p, 16, with_w=False)
    pooled = _sc_poolmax(ms_new, idx_e)
    ms_edge = _edge_mm(pooled, ms_new, W_edge)

    x_out = _dense_relu([d0, ms_edge],
                        [W_ee[:d0_dim], W_ee[d0_dim:]])
    del n0
    return _classifier(x_out, W_cls1, b_cls1, gamma, beta, W_cls2, b_cls2)


# speedup vs baseline: 5.9909x; 1.0593x over previous
import functools

import jax
import jax.numpy as jnp
from jax import lax
from jax.experimental import pallas as pl
from jax.experimental.pallas import tpu as pltpu
from jax.experimental.pallas import tpu_sc as plsc

_BIG = 3.0e38


def _knn_body(k, with_w, nr, q_ref, r_ref, idx_ref, *maybe_w_ref):
    q = q_ref[...]
    r = r_ref[...]
    qn = jnp.sum(q * q, axis=1, keepdims=True)
    rn = jnp.sum(r * r, axis=1)[None, :]
    qr = lax.dot_general(q, r, (((1,), (1,)), ((), ())),
                         preferred_element_type=jnp.float32)
    d = qn - 2.0 * qr + rn
    iota = lax.broadcasted_iota(jnp.int32, d.shape, 1)
    cols = []
    vals = []
    for _ in range(k):
        m = jnp.min(d, axis=1, keepdims=True)
        cand = jnp.where(d == m, iota, nr)
        col = jnp.min(cand, axis=1, keepdims=True)
        cols.append(col)
        vals.append(m)
        d = jnp.where(iota == col, _BIG, d)
    idx_ref[...] = jnp.concatenate(cols, axis=1)
    if with_w:
        d2 = jnp.maximum(jnp.concatenate(vals, axis=1), 0.0)
        w = 1.0 / (d2 + 1e-8)
        maybe_w_ref[0][...] = w / jnp.sum(w, axis=1, keepdims=True)


def _knn(q, r, k, with_w, block_rows=128):
    nq = q.shape[0]
    nr = r.shape[0]
    bq = min(block_rows, nq)
    grid = (nq // bq,)
    out_shape = [jax.ShapeDtypeStruct((nq, k), jnp.int32)]
    out_specs = [pl.BlockSpec((bq, k), lambda i: (i, 0))]
    if with_w:
        out_shape.append(jax.ShapeDtypeStruct((nq, k), jnp.float32))
        out_specs.append(pl.BlockSpec((bq, k), lambda i: (i, 0)))
    res = pl.pallas_call(
        functools.partial(_knn_body, k, with_w, nr),
        grid=grid,
        in_specs=[pl.BlockSpec((bq, 3), lambda i: (i, 0)),
                  pl.BlockSpec((nr, 3), lambda i: (0, 0))],
        out_specs=out_specs,
        out_shape=out_shape,
    )(q, r)
    return res if with_w else res[0]


def _dense_body(n_in, *refs):
    out_ref = refs[-1]
    acc = None
    for i in range(n_in):
        part = jnp.dot(refs[i][...], refs[n_in + i][...],
                       preferred_element_type=jnp.float32)
        acc = part if acc is None else acc + part
    out_ref[...] = jnp.maximum(acc, 0.0)


def _dense_relu(xs, ws, block_rows=512):
    m = xs[0].shape[0]
    bm = min(block_rows, m)
    n = ws[0].shape[1]
    in_specs = []
    for x in xs:
        kd = x.shape[1]
        in_specs.append(pl.BlockSpec((bm, kd), lambda i: (i, 0)))
    for w in ws:
        in_specs.append(pl.BlockSpec(w.shape, lambda i: (0, 0)))
    return pl.pallas_call(
        functools.partial(_dense_body, len(xs)),
        grid=(m // bm,),
        in_specs=in_specs,
        out_specs=pl.BlockSpec((bm, n), lambda i: (i, 0)),
        out_shape=jax.ShapeDtypeStruct((m, n), jnp.float32),
    )(*xs, *ws)


def _edge_body(a_ref, b_ref, w_ref, o_ref):
    o_ref[...] = jnp.maximum(
        jnp.dot(a_ref[...] - b_ref[...], w_ref[...],
                preferred_element_type=jnp.float32), 0.0)


def _edge_mm(a, b, w, block_rows=512):
    m, kd = a.shape
    n = w.shape[1]
    bm = min(block_rows, m)
    return pl.pallas_call(
        _edge_body,
        grid=(m // bm,),
        in_specs=[pl.BlockSpec((bm, kd), lambda i: (i, 0)),
                  pl.BlockSpec((bm, kd), lambda i: (i, 0)),
                  pl.BlockSpec(w.shape, lambda i: (0, 0))],
        out_specs=pl.BlockSpec((bm, n), lambda i: (i, 0)),
        out_shape=jax.ShapeDtypeStruct((m, n), jnp.float32),
    )(a, b, w)


def _cls_body(x_ref, w1_ref, b1_ref, g_ref, be_ref, w2_ref, b2_ref, o_ref):
    h = jnp.dot(x_ref[...], w1_ref[...],
                preferred_element_type=jnp.float32) + b1_ref[...]
    mu = jnp.mean(h, axis=0, keepdims=True)
    var = jnp.mean((h - mu) * (h - mu), axis=0, keepdims=True)
    h = g_ref[...] * (h - mu) / jnp.sqrt(var + 1e-5) + be_ref[...]
    h = jnp.maximum(h, 0.0)
    o_ref[...] = jnp.dot(h, w2_ref[...],
                         preferred_element_type=jnp.float32) + b2_ref[...]


def _classifier(x, w1, b1, g, be, w2, b2):
    m, d = x.shape
    n = w2.shape[1]
    full = lambda s: pl.BlockSpec(s, lambda: tuple(0 for _ in s))
    return pl.pallas_call(
        _cls_body,
        in_specs=[full((m, d)), full(w1.shape), full((1, d)), full((1, d)),
                  full((1, d)), full(w2.shape), full((1, n))],
        out_specs=full((m, n)),
        out_shape=jax.ShapeDtypeStruct((m, n), jnp.float32),
    )(x, w1, b1.reshape(1, -1), g.reshape(1, -1), be.reshape(1, -1),
      w2, b2.reshape(1, -1))


def _pad128(table):
    d = table.shape[1]
    dp = (d + 127) // 128 * 128
    if dp != d:
        table = jnp.pad(table, ((0, 0), (0, dp - d)))
    return table, d


def _sc_gather_reduce(table, idx, w):
    with_w = w is not None
    table, d_orig = _pad128(table)
    q_tot, kk = idx.shape
    _, d = table.shape
    info = plsc.get_sparse_core_info()
    nw = info.num_cores * info.num_subcores
    qpw = q_tot // nw
    b = max(1, min(128 // kk, qpw))
    nqb = qpw // b
    mesh = plsc.VectorSubcoreMesh(core_axis_name="c", subcore_axis_name="s")

    scratch = [pltpu.VMEM((qpw * kk,), jnp.int32)]
    if with_w:
        scratch.append(pltpu.VMEM((qpw * kk + 16,), jnp.float32))
    scratch += [
        pltpu.VMEM((b * kk, d), jnp.float32),
        pltpu.VMEM((b * kk, d), jnp.float32),
        pltpu.VMEM((b, d), jnp.float32),
        pltpu.VMEM((b, d), jnp.float32),
        pltpu.SemaphoreType.DMA,
        pltpu.SemaphoreType.DMA,
        pltpu.SemaphoreType.DMA,
        pltpu.SemaphoreType.DMA,
    ]

    @functools.partial(
        pl.kernel, mesh=mesh,
        out_type=jax.ShapeDtypeStruct((q_tot, d), jnp.float32),
        scratch_types=scratch,
    )
    def kern(*refs):
        if with_w:
            (table_hbm, idx_hbm, w_hbm, out_hbm, idx_v, w_v,
             rows0, rows1, out0, out1, sg0, sg1, ss0, ss1) = refs
        else:
            (table_hbm, idx_hbm, out_hbm, idx_v,
             rows0, rows1, out0, out1, sg0, sg1, ss0, ss1) = refs
        rows = (rows0, rows1)
        outs = (out0, out1)
        sgs = (sg0, sg1)
        sss = (ss0, ss1)
        wid = lax.axis_index("s") * info.num_cores + lax.axis_index("c")
        base = wid * qpw
        pltpu.sync_copy(idx_hbm.at[pl.ds(base * kk, qpw * kk)], idx_v)
        if with_w:
            pltpu.sync_copy(w_hbm.at[pl.ds(base * kk, qpw * kk)],
                            w_v.at[pl.ds(0, qpw * kk)])

        def gather(qb, s):
            return pltpu.make_async_copy(
                table_hbm.at[idx_v.at[pl.ds(qb * b * kk, b * kk)]],
                rows[s], sgs[s])

        def store(qb, s):
            return pltpu.make_async_copy(
                outs[s], out_hbm.at[pl.ds(base + qb * b, b)], sss[s])

        def compute(qb, s):
            rv, ov = rows[s], outs[s]

            def b_body(bi):
                if with_w:
                    wvec = w_v[pl.ds((qb * b + bi) * kk, 16)]
                    wjs = [wvec[j] for j in range(kk)]
                for dch in range(d // 16):
                    sl = pl.ds(dch * 16, 16)
                    if with_w:
                        acc = wjs[0] * rv[bi * kk, sl]
                        for j in range(1, kk):
                            acc = acc + wjs[j] * rv[bi * kk + j, sl]
                    else:
                        acc = rv[bi * kk, sl]
                        for j in range(1, kk):
                            acc = jnp.maximum(acc, rv[bi * kk + j, sl])
                    ov[bi, sl] = acc
            lax.fori_loop(0, b, lambda i, _: (b_body(i), 0)[1], 0)

        gather(0, 0).start()
        nqb2 = nqb - (nqb % 2)

        if nqb2 >= 2:
            @pl.loop(0, nqb2, step=2)
            def _(qb):
                gather(qb + 1, 1).start()
                gather(qb, 0).wait()

                @pl.when(qb >= 2)
                def _():
                    store(qb - 2, 0).wait()
                compute(qb, 0)
                store(qb, 0).start()

                @pl.when(qb + 2 < nqb)
                def _():
                    gather(qb + 2, 0).start()
                gather(qb + 1, 1).wait()

                @pl.when(qb >= 2)
                def _():
                    store(qb - 1, 1).wait()
                compute(qb + 1, 1)
                store(qb + 1, 1).start()

        if nqb % 2:
            last = nqb - 1
            gather(last, 0).wait()
            if nqb2 >= 2:
                store(nqb2 - 2, 0).wait()
            compute(last, 0)
            store(last, 0).start()
            store(last, 0).wait()
            if nqb2 >= 2:
                store(nqb2 - 1, 1).wait()
        else:
            store(nqb - 2, 0).wait()
            store(nqb - 1, 1).wait()

    if with_w:
        out = kern(table, idx.reshape(-1), w.reshape(-1))
    else:
        out = kern(table, idx.reshape(-1))
    return out[:, :d_orig] if d_orig != d else out


def _sc_poolmax(table, idx):
    return _sc_gather_reduce(table, idx, None)


def _sc_interp(table, idx, w):
    return _sc_gather_reduce(table, idx, w)


def kernel(p, x, W_enc0, W_enc1, W_enc2, W_dec1, W_dec0, W_fusion, W_edge,
           W_ee, W_cls1, b_cls1, gamma, beta, W_cls2, b_cls2, o):
    del o
    n0 = p.shape[0]
    d0_dim, d1_dim, d2_dim = W_enc0.shape[1], W_enc1.shape[1], W_enc2.shape[1]

    x0 = _dense_relu([x], [W_enc0])
    p1 = p[::4]
    idx1 = _knn(p1, p, 16, with_w=False)
    x1 = _dense_relu([_sc_poolmax(x0, idx1)], [W_enc1])
    p2 = p1[::4]
    idx2 = _knn(p2, p1, 16, with_w=False)
    x2 = _dense_relu([_sc_poolmax(x1, idx2)], [W_enc2])

    idx_u1, w_u1 = _knn(p1, p2, 8, with_w=True)
    up1 = _sc_interp(x2, idx_u1, w_u1)
    d1 = _dense_relu([up1, x1],
                     [W_dec1[:d2_dim], W_dec1[d2_dim:]])
    idx_u0, w_u0 = _knn(p, p1, 8, with_w=True)
    up0 = _sc_interp(d1, idx_u0, w_u0)
    d0 = _dense_relu([up0, x0],
                     [W_dec0[:d1_dim], W_dec0[d1_dim:]])

    ms1 = _sc_interp(x1, idx_u0, w_u0)
    idx_m2, w_m2 = _knn(p, p2, 8, with_w=True)
    ms2 = _sc_interp(x2, idx_m2, w_m2)
    ws = [W_fusion[:d0_dim], W_fusion[d0_dim:d0_dim + d1_dim],
          W_fusion[d0_dim + d1_dim:]]
    ms_new = _dense_relu([x0, ms1, ms2], ws)

    idx_e = _knn(p, p, 16, with_w=False)
    pooled = _sc_poolmax(ms_new, idx_e)
    ms_edge = _edge_mm(pooled, ms_new, W_edge)

    x_out = _dense_relu([d0, ms_edge],
                        [W_ee[:d0_dim], W_ee[d0_dim:]])
    del n0
    return _classifier(x_out, W_cls1, b_cls1, gamma, beta, W_cls2, b_cls2)


# speedup vs baseline: 6.8516x; 1.1437x over previous
import functools

import jax
import jax.numpy as jnp
from jax import lax
from jax.experimental import pallas as pl
from jax.experimental.pallas import tpu as pltpu
from jax.experimental.pallas import tpu_sc as plsc

_BIG = 3.0e38


def _knn_body(k, with_w, nr, q_ref, r_ref, idx_ref, *maybe_w_ref):
    q = q_ref[...]
    r = r_ref[...]
    qn = jnp.sum(q * q, axis=1, keepdims=True)
    rn = jnp.sum(r * r, axis=1)[None, :]
    qr = lax.dot_general(q, r, (((1,), (1,)), ((), ())),
                         preferred_element_type=jnp.float32)
    d = qn - 2.0 * qr + rn
    iota = lax.broadcasted_iota(jnp.int32, d.shape, 1)
    cols = []
    vals = []
    for _ in range(k):
        m = jnp.min(d, axis=1, keepdims=True)
        cand = jnp.where(d == m, iota, nr)
        col = jnp.min(cand, axis=1, keepdims=True)
        cols.append(col)
        vals.append(m)
        d = jnp.where(iota == col, _BIG, d)
    idx_ref[...] = jnp.concatenate(cols, axis=1)
    if with_w:
        d2 = jnp.maximum(jnp.concatenate(vals, axis=1), 0.0)
        w = 1.0 / (d2 + 1e-8)
        maybe_w_ref[0][...] = w / jnp.sum(w, axis=1, keepdims=True)


def _knn(q, r, k, with_w, block_rows=128):
    nq = q.shape[0]
    nr = r.shape[0]
    bq = min(block_rows, nq)
    grid = (nq // bq,)
    out_shape = [jax.ShapeDtypeStruct((nq, k), jnp.int32)]
    out_specs = [pl.BlockSpec((bq, k), lambda i: (i, 0))]
    if with_w:
        out_shape.append(jax.ShapeDtypeStruct((nq, k), jnp.float32))
        out_specs.append(pl.BlockSpec((bq, k), lambda i: (i, 0)))
    res = pl.pallas_call(
        functools.partial(_knn_body, k, with_w, nr),
        grid=grid,
        in_specs=[pl.BlockSpec((bq, 3), lambda i: (i, 0)),
                  pl.BlockSpec((nr, 3), lambda i: (0, 0))],
        out_specs=out_specs,
        out_shape=out_shape,
    )(q, r)
    return res if with_w else res[0]


def _dense_body(n_in, *refs):
    out_ref = refs[-1]
    acc = None
    for i in range(n_in):
        part = jnp.dot(refs[i][...], refs[n_in + i][...],
                       preferred_element_type=jnp.float32)
        acc = part if acc is None else acc + part
    out_ref[...] = jnp.maximum(acc, 0.0)


def _dense_relu(xs, ws, block_rows=512):
    m = xs[0].shape[0]
    bm = min(block_rows, m)
    n = ws[0].shape[1]
    in_specs = []
    for x in xs:
        kd = x.shape[1]
        in_specs.append(pl.BlockSpec((bm, kd), lambda i: (i, 0)))
    for w in ws:
        in_specs.append(pl.BlockSpec(w.shape, lambda i: (0, 0)))
    return pl.pallas_call(
        functools.partial(_dense_body, len(xs)),
        grid=(m // bm,),
        in_specs=in_specs,
        out_specs=pl.BlockSpec((bm, n), lambda i: (i, 0)),
        out_shape=jax.ShapeDtypeStruct((m, n), jnp.float32),
    )(*xs, *ws)


def _edge_body(a_ref, b_ref, w_ref, o_ref):
    o_ref[...] = jnp.maximum(
        jnp.dot(a_ref[...] - b_ref[...], w_ref[...],
                preferred_element_type=jnp.float32), 0.0)


def _edge_mm(a, b, w, block_rows=512):
    m, kd = a.shape
    n = w.shape[1]
    bm = min(block_rows, m)
    return pl.pallas_call(
        _edge_body,
        grid=(m // bm,),
        in_specs=[pl.BlockSpec((bm, kd), lambda i: (i, 0)),
                  pl.BlockSpec((bm, kd), lambda i: (i, 0)),
                  pl.BlockSpec(w.shape, lambda i: (0, 0))],
        out_specs=pl.BlockSpec((bm, n), lambda i: (i, 0)),
        out_shape=jax.ShapeDtypeStruct((m, n), jnp.float32),
    )(a, b, w)


def _cls_body(x_ref, w1_ref, b1_ref, g_ref, be_ref, w2_ref, b2_ref, o_ref):
    h = jnp.dot(x_ref[...], w1_ref[...],
                preferred_element_type=jnp.float32) + b1_ref[...]
    mu = jnp.mean(h, axis=0, keepdims=True)
    var = jnp.mean((h - mu) * (h - mu), axis=0, keepdims=True)
    h = g_ref[...] * (h - mu) / jnp.sqrt(var + 1e-5) + be_ref[...]
    h = jnp.maximum(h, 0.0)
    o_ref[...] = jnp.dot(h, w2_ref[...],
                         preferred_element_type=jnp.float32) + b2_ref[...]


def _classifier(x, w1, b1, g, be, w2, b2):
    m, d = x.shape
    n = w2.shape[1]
    full = lambda s: pl.BlockSpec(s, lambda: tuple(0 for _ in s))
    return pl.pallas_call(
        _cls_body,
        in_specs=[full((m, d)), full(w1.shape), full((1, d)), full((1, d)),
                  full((1, d)), full(w2.shape), full((1, n))],
        out_specs=full((m, n)),
        out_shape=jax.ShapeDtypeStruct((m, n), jnp.float32),
    )(x, w1, b1.reshape(1, -1), g.reshape(1, -1), be.reshape(1, -1),
      w2, b2.reshape(1, -1))


def _pad128(table):
    d = table.shape[1]
    dp = (d + 127) // 128 * 128
    if dp != d:
        table = jnp.pad(table, ((0, 0), (0, dp - d)))
    return table, d


def _sc_gather_reduce(table, idx, w):
    with_w = w is not None
    table, d_orig = _pad128(table)
    q_tot, kk = idx.shape
    _, d = table.shape
    info = plsc.get_sparse_core_info()
    nw = info.num_cores * info.num_subcores
    qpw = q_tot // nw
    b = max(1, min(128 // kk, qpw))
    nqb = qpw // b
    mesh = plsc.VectorSubcoreMesh(core_axis_name="c", subcore_axis_name="s")

    scratch = [pltpu.VMEM((qpw * kk,), jnp.int32)]
    if with_w:
        scratch.append(pltpu.VMEM((qpw * kk + 16,), jnp.float32))
    scratch += [
        pltpu.VMEM((b * kk, d), jnp.float32),
        pltpu.VMEM((b * kk, d), jnp.float32),
        pltpu.VMEM((b, d), jnp.float32),
        pltpu.VMEM((b, d), jnp.float32),
        pltpu.SemaphoreType.DMA,
        pltpu.SemaphoreType.DMA,
        pltpu.SemaphoreType.DMA,
        pltpu.SemaphoreType.DMA,
    ]

    @functools.partial(
        pl.kernel, mesh=mesh,
        out_type=jax.ShapeDtypeStruct((q_tot, d), jnp.float32),
        scratch_types=scratch,
    )
    def kern(*refs):
        if with_w:
            (table_hbm, idx_hbm, w_hbm, out_hbm, idx_v, w_v,
             rows0, rows1, out0, out1, sg0, sg1, ss0, ss1) = refs
        else:
            (table_hbm, idx_hbm, out_hbm, idx_v,
             rows0, rows1, out0, out1, sg0, sg1, ss0, ss1) = refs
        rows = (rows0, rows1)
        outs = (out0, out1)
        sgs = (sg0, sg1)
        sss = (ss0, ss1)
        wid = lax.axis_index("s") * info.num_cores + lax.axis_index("c")
        base = wid * qpw
        pltpu.sync_copy(idx_hbm.at[pl.ds(base * kk, qpw * kk)], idx_v)
        if with_w:
            pltpu.sync_copy(w_hbm.at[pl.ds(base * kk, qpw * kk)],
                            w_v.at[pl.ds(0, qpw * kk)])

        def gather(qb, s):
            return pltpu.make_async_copy(
                table_hbm.at[idx_v.at[pl.ds(qb * b * kk, b * kk)]],
                rows[s], sgs[s])

        def store(qb, s):
            return pltpu.make_async_copy(
                outs[s], out_hbm.at[pl.ds(base + qb * b, b)], sss[s])

        def compute(qb, s):
            rv, ov = rows[s], outs[s]

            def b_body(bi):
                if with_w:
                    wvec = w_v[pl.ds((qb * b + bi) * kk, 16)]
                    wjs = [wvec[j] for j in range(kk)]
                for dch in range(d // 16):
                    sl = pl.ds(dch * 16, 16)
                    if with_w:
                        acc = wjs[0] * rv[bi * kk, sl]
                        for j in range(1, kk):
                            acc = acc + wjs[j] * rv[bi * kk + j, sl]
                    else:
                        acc = rv[bi * kk, sl]
                        for j in range(1, kk):
                            acc = jnp.maximum(acc, rv[bi * kk + j, sl])
                    ov[bi, sl] = acc
            lax.fori_loop(0, b, lambda i, _: (b_body(i), 0)[1], 0)

        gather(0, 0).start()
        nqb2 = nqb - (nqb % 2)

        if nqb2 >= 2:
            @pl.loop(0, nqb2, step=2)
            def _(qb):
                gather(qb + 1, 1).start()
                gather(qb, 0).wait()

                @pl.when(qb >= 2)
                def _():
                    store(qb - 2, 0).wait()
                compute(qb, 0)
                store(qb, 0).start()

                @pl.when(qb + 2 < nqb)
                def _():
                    gather(qb + 2, 0).start()
                gather(qb + 1, 1).wait()

                @pl.when(qb >= 2)
                def _():
                    store(qb - 1, 1).wait()
                compute(qb + 1, 1)
                store(qb + 1, 1).start()

        if nqb % 2:
            last = nqb - 1
            gather(last, 0).wait()
            if nqb2 >= 2:
                store(nqb2 - 2, 0).wait()
            compute(last, 0)
            store(last, 0).start()
            store(last, 0).wait()
            if nqb2 >= 2:
                store(nqb2 - 1, 1).wait()
        else:
            store(nqb - 2, 0).wait()
            store(nqb - 1, 1).wait()

    if with_w:
        out = kern(table, idx.reshape(-1), w.reshape(-1))
    else:
        out = kern(table, idx.reshape(-1))
    return out[:, :d_orig] if d_orig != d else out


def _sc_poolmax(table, idx):
    return _sc_gather_reduce(table, idx, None)


def _sc_interp(table, idx, w):
    return _sc_gather_reduce(table, idx, w)


def kernel(p, x, W_enc0, W_enc1, W_enc2, W_dec1, W_dec0, W_fusion, W_edge,
           W_ee, W_cls1, b_cls1, gamma, beta, W_cls2, b_cls2, o):
    del o
    n0 = p.shape[0]
    d0_dim, d1_dim, d2_dim = W_enc0.shape[1], W_enc1.shape[1], W_enc2.shape[1]

    p1 = p[::4]
    p2 = p1[::4]
    idx_e = _knn(p, p, 16, with_w=False)
    idx1 = idx_e[::4]
    idx2 = _knn(p2, p1, 16, with_w=False)
    idx_m2, w_m2 = _knn(p, p2, 8, with_w=True)
    idx_u1, w_u1 = idx_m2[::4], w_m2[::4]
    idx_u0, w_u0 = _knn(p, p1, 8, with_w=True)

    x0 = _dense_relu([x], [W_enc0])
    x1 = _dense_relu([_sc_poolmax(x0, idx1)], [W_enc1])
    x2 = _dense_relu([_sc_poolmax(x1, idx2)], [W_enc2])

    up1 = _sc_interp(x2, idx_u1, w_u1)
    d1 = _dense_relu([up1, x1],
                     [W_dec1[:d2_dim], W_dec1[d2_dim:]])
    up0 = _sc_interp(d1, idx_u0, w_u0)
    d0 = _dense_relu([up0, x0],
                     [W_dec0[:d1_dim], W_dec0[d1_dim:]])

    ms1 = _sc_interp(x1, idx_u0, w_u0)
    ms2 = _sc_interp(x2, idx_m2, w_m2)
    ws = [W_fusion[:d0_dim], W_fusion[d0_dim:d0_dim + d1_dim],
          W_fusion[d0_dim + d1_dim:]]
    ms_new = _dense_relu([x0, ms1, ms2], ws)

    pooled = _sc_poolmax(ms_new, idx_e)
    ms_edge = _edge_mm(pooled, ms_new, W_edge)

    x_out = _dense_relu([d0, ms_edge],
                        [W_ee[:d0_dim], W_ee[d0_dim:]])
    del n0
    return _classifier(x_out, W_cls1, b_cls1, gamma, beta, W_cls2, b_cls2)


# speedup vs baseline: 7.6641x; 1.1186x over previous
import functools

import jax
import jax.numpy as jnp
from jax import lax
from jax.experimental import pallas as pl
from jax.experimental.pallas import tpu as pltpu
from jax.experimental.pallas import tpu_sc as plsc

_BIG = 3.0e38


def _aug_body(r_ref, o_ref):
    r = r_ref[...]
    o_ref[...] = jnp.concatenate(
        [r, jnp.sum(r * r, axis=1, keepdims=True)], axis=1)


def _augment(r):
    nr = r.shape[0]
    return pl.pallas_call(
        _aug_body,
        in_specs=[pl.BlockSpec((nr, 3), lambda: (0, 0))],
        out_specs=pl.BlockSpec((nr, 4), lambda: (0, 0)),
        out_shape=jax.ShapeDtypeStruct((nr, 4), jnp.float32),
    )(r)


def _knn_body(k, with_w, nr, q_ref, r4_ref, idx_ref, *maybe_w_ref):
    q = q_ref[...]
    r4 = r4_ref[...]
    qn = jnp.sum(q * q, axis=1, keepdims=True)
    r = r4[:, :3]
    rn = jnp.sum(r * r, axis=1)[None, :]
    qr = lax.dot_general(q, r, (((1,), (1,)), ((), ())),
                         preferred_element_type=jnp.float32)
    d = qn - 2.0 * qr + rn
    iota = lax.broadcasted_iota(jnp.int32, d.shape, 1).astype(jnp.float32)
    nrf = float(nr)
    cols = []
    vals = []
    for _ in range(k):
        m = jnp.min(d, axis=1, keepdims=True)
        cand = jnp.where(d == m, iota, nrf)
        col = jnp.min(cand, axis=1, keepdims=True)
        cols.append(col)
        vals.append(m)
        d = jnp.where(cand == col, _BIG, d)
    idx_ref[...] = jnp.concatenate(cols, axis=1).astype(jnp.int32)
    if with_w:
        d2 = jnp.maximum(jnp.concatenate(vals, axis=1), 0.0)
        w = 1.0 / (d2 + 1e-8)
        maybe_w_ref[0][...] = w / jnp.sum(w, axis=1, keepdims=True)


def _knn(q, r4, k, with_w, block_rows=128):
    nq = q.shape[0]
    nr = r4.shape[0]
    bq = min(block_rows, nq)
    grid = (nq // bq,)
    out_shape = [jax.ShapeDtypeStruct((nq, k), jnp.int32)]
    out_specs = [pl.BlockSpec((bq, k), lambda i: (i, 0))]
    if with_w:
        out_shape.append(jax.ShapeDtypeStruct((nq, k), jnp.float32))
        out_specs.append(pl.BlockSpec((bq, k), lambda i: (i, 0)))
    res = pl.pallas_call(
        functools.partial(_knn_body, k, with_w, nr),
        grid=grid,
        in_specs=[pl.BlockSpec((bq, 3), lambda i: (i, 0)),
                  pl.BlockSpec((nr, 4), lambda i: (0, 0))],
        out_specs=out_specs,
        out_shape=out_shape,
    )(q, r4)
    return res if with_w else res[0]


def _dense_body(n_in, *refs):
    out_ref = refs[-1]
    acc = None
    for i in range(n_in):
        part = jnp.dot(refs[i][...], refs[n_in + i][...],
                       preferred_element_type=jnp.float32)
        acc = part if acc is None else acc + part
    out_ref[...] = jnp.maximum(acc, 0.0)


def _dense_relu(xs, ws, block_rows=512):
    m = xs[0].shape[0]
    bm = min(block_rows, m)
    n = ws[0].shape[1]
    in_specs = []
    for x in xs:
        kd = x.shape[1]
        in_specs.append(pl.BlockSpec((bm, kd), lambda i: (i, 0)))
    for w in ws:
        in_specs.append(pl.BlockSpec(w.shape, lambda i: (0, 0)))
    return pl.pallas_call(
        functools.partial(_dense_body, len(xs)),
        grid=(m // bm,),
        in_specs=in_specs,
        out_specs=pl.BlockSpec((bm, n), lambda i: (i, 0)),
        out_shape=jax.ShapeDtypeStruct((m, n), jnp.float32),
    )(*xs, *ws)


def _edge_body(a_ref, b_ref, w_ref, o_ref):
    o_ref[...] = jnp.maximum(
        jnp.dot(a_ref[...] - b_ref[...], w_ref[...],
                preferred_element_type=jnp.float32), 0.0)


def _edge_mm(a, b, w, block_rows=512):
    m, kd = a.shape
    n = w.shape[1]
    bm = min(block_rows, m)
    return pl.pallas_call(
        _edge_body,
        grid=(m // bm,),
        in_specs=[pl.BlockSpec((bm, kd), lambda i: (i, 0)),
                  pl.BlockSpec((bm, kd), lambda i: (i, 0)),
                  pl.BlockSpec(w.shape, lambda i: (0, 0))],
        out_specs=pl.BlockSpec((bm, n), lambda i: (i, 0)),
        out_shape=jax.ShapeDtypeStruct((m, n), jnp.float32),
    )(a, b, w)


def _cls_body(x_ref, w1_ref, b1_ref, g_ref, be_ref, w2_ref, b2_ref, o_ref):
    h = jnp.dot(x_ref[...], w1_ref[...],
                preferred_element_type=jnp.float32) + b1_ref[...]
    mu = jnp.mean(h, axis=0, keepdims=True)
    var = jnp.mean((h - mu) * (h - mu), axis=0, keepdims=True)
    h = g_ref[...] * (h - mu) / jnp.sqrt(var + 1e-5) + be_ref[...]
    h = jnp.maximum(h, 0.0)
    o_ref[...] = jnp.dot(h, w2_ref[...],
                         preferred_element_type=jnp.float32) + b2_ref[...]


def _classifier(x, w1, b1, g, be, w2, b2):
    m, d = x.shape
    n = w2.shape[1]
    full = lambda s: pl.BlockSpec(s, lambda: tuple(0 for _ in s))
    return pl.pallas_call(
        _cls_body,
        in_specs=[full((m, d)), full(w1.shape), full((1, d)), full((1, d)),
                  full((1, d)), full(w2.shape), full((1, n))],
        out_specs=full((m, n)),
        out_shape=jax.ShapeDtypeStruct((m, n), jnp.float32),
    )(x, w1, b1.reshape(1, -1), g.reshape(1, -1), be.reshape(1, -1),
      w2, b2.reshape(1, -1))


def _pad128(table):
    d = table.shape[1]
    dp = (d + 127) // 128 * 128
    if dp != d:
        table = jnp.pad(table, ((0, 0), (0, dp - d)))
    return table, d


def _sc_gather_reduce(table, idx, w):
    with_w = w is not None
    table, d_orig = _pad128(table)
    q_tot, kk = idx.shape
    _, d = table.shape
    info = plsc.get_sparse_core_info()
    nw = info.num_cores * info.num_subcores
    qpw = q_tot // nw
    b = max(1, min(128 // kk, qpw))
    nqb = qpw // b
    mesh = plsc.VectorSubcoreMesh(core_axis_name="c", subcore_axis_name="s")

    scratch = [pltpu.VMEM((qpw * kk,), jnp.int32)]
    if with_w:
        scratch.append(pltpu.VMEM((qpw * kk + 16,), jnp.float32))
    scratch += [
        pltpu.VMEM((b * kk, d), jnp.float32),
        pltpu.VMEM((b * kk, d), jnp.float32),
        pltpu.VMEM((b, d), jnp.float32),
        pltpu.VMEM((b, d), jnp.float32),
        pltpu.SemaphoreType.DMA,
        pltpu.SemaphoreType.DMA,
        pltpu.SemaphoreType.DMA,
        pltpu.SemaphoreType.DMA,
    ]

    @functools.partial(
        pl.kernel, mesh=mesh,
        out_type=jax.ShapeDtypeStruct((q_tot, d), jnp.float32),
        scratch_types=scratch,
    )
    def kern(*refs):
        if with_w:
            (table_hbm, idx_hbm, w_hbm, out_hbm, idx_v, w_v,
             rows0, rows1, out0, out1, sg0, sg1, ss0, ss1) = refs
        else:
            (table_hbm, idx_hbm, out_hbm, idx_v,
             rows0, rows1, out0, out1, sg0, sg1, ss0, ss1) = refs
        rows = (rows0, rows1)
        outs = (out0, out1)
        sgs = (sg0, sg1)
        sss = (ss0, ss1)
        wid = lax.axis_index("s") * info.num_cores + lax.axis_index("c")
        base = wid * qpw
        pltpu.sync_copy(idx_hbm.at[pl.ds(base * kk, qpw * kk)], idx_v)
        if with_w:
            pltpu.sync_copy(w_hbm.at[pl.ds(base * kk, qpw * kk)],
                            w_v.at[pl.ds(0, qpw * kk)])

        def gather(qb, s):
            return pltpu.make_async_copy(
                table_hbm.at[idx_v.at[pl.ds(qb * b * kk, b * kk)]],
                rows[s], sgs[s])

        def store(qb, s):
            return pltpu.make_async_copy(
                outs[s], out_hbm.at[pl.ds(base + qb * b, b)], sss[s])

        def compute(qb, s):
            rv, ov = rows[s], outs[s]

            def b_body(bi):
                if with_w:
                    wvec = w_v[pl.ds((qb * b + bi) * kk, 16)]
                    wjs = [wvec[j] for j in range(kk)]
                for dch in range(d // 16):
                    sl = pl.ds(dch * 16, 16)
                    if with_w:
                        acc = wjs[0] * rv[bi * kk, sl]
                        for j in range(1, kk):
                            acc = acc + wjs[j] * rv[bi * kk + j, sl]
                    else:
                        acc = rv[bi * kk, sl]
                        for j in range(1, kk):
                            acc = jnp.maximum(acc, rv[bi * kk + j, sl])
                    ov[bi, sl] = acc
            lax.fori_loop(0, b, lambda i, _: (b_body(i), 0)[1], 0)

        gather(0, 0).start()
        nqb2 = nqb - (nqb % 2)

        if nqb2 >= 2:
            @pl.loop(0, nqb2, step=2)
            def _(qb):
                gather(qb + 1, 1).start()
                gather(qb, 0).wait()

                @pl.when(qb >= 2)
                def _():
                    store(qb - 2, 0).wait()
                compute(qb, 0)
                store(qb, 0).start()

                @pl.when(qb + 2 < nqb)
                def _():
                    gather(qb + 2, 0).start()
                gather(qb + 1, 1).wait()

                @pl.when(qb >= 2)
                def _():
                    store(qb - 1, 1).wait()
                compute(qb + 1, 1)
                store(qb + 1, 1).start()

        if nqb % 2:
            last = nqb - 1
            gather(last, 0).wait()
            if nqb2 >= 2:
                store(nqb2 - 2, 0).wait()
            compute(last, 0)
            store(last, 0).start()
            store(last, 0).wait()
            if nqb2 >= 2:
                store(nqb2 - 1, 1).wait()
        else:
            store(nqb - 2, 0).wait()
            store(nqb - 1, 1).wait()

    if with_w:
        out = kern(table, idx.reshape(-1), w.reshape(-1))
    else:
        out = kern(table, idx.reshape(-1))
    return out[:, :d_orig] if d_orig != d else out


def _sc_poolmax(table, idx):
    return _sc_gather_reduce(table, idx, None)


def _sc_interp(table, idx, w):
    return _sc_gather_reduce(table, idx, w)


def kernel(p, x, W_enc0, W_enc1, W_enc2, W_dec1, W_dec0, W_fusion, W_edge,
           W_ee, W_cls1, b_cls1, gamma, beta, W_cls2, b_cls2, o):
    del o
    n0 = p.shape[0]
    d0_dim, d1_dim, d2_dim = W_enc0.shape[1], W_enc1.shape[1], W_enc2.shape[1]

    p1 = p[::4]
    p2 = p1[::4]
    p_a, p1_a, p2_a = _augment(p), _augment(p1), _augment(p2)
    idx_e = _knn(p, p_a, 16, with_w=False)
    idx1 = idx_e[::4]
    idx2 = _knn(p2, p1_a, 16, with_w=False)
    idx_m2, w_m2 = _knn(p, p2_a, 8, with_w=True)
    idx_u1, w_u1 = idx_m2[::4], w_m2[::4]
    idx_u0, w_u0 = _knn(p, p1_a, 8, with_w=True)

    x0 = _dense_relu([x], [W_enc0])
    x1 = _dense_relu([_sc_poolmax(x0, idx1)], [W_enc1])
    x2 = _dense_relu([_sc_poolmax(x1, idx2)], [W_enc2])

    up1 = _sc_interp(x2, idx_u1, w_u1)
    d1 = _dense_relu([up1, x1],
                     [W_dec1[:d2_dim], W_dec1[d2_dim:]])
    up0 = _sc_interp(d1, idx_u0, w_u0)
    d0 = _dense_relu([up0, x0],
                     [W_dec0[:d1_dim], W_dec0[d1_dim:]])

    ms1 = _sc_interp(x1, idx_u0, w_u0)
    ms2 = _sc_interp(x2, idx_m2, w_m2)
    ws = [W_fusion[:d0_dim], W_fusion[d0_dim:d0_dim + d1_dim],
          W_fusion[d0_dim + d1_dim:]]
    ms_new = _dense_relu([x0, ms1, ms2], ws)

    pooled = _sc_poolmax(ms_new, idx_e)
    ms_edge = _edge_mm(pooled, ms_new, W_edge)

    x_out = _dense_relu([d0, ms_edge],
                        [W_ee[:d0_dim], W_ee[d0_dim:]])
    del n0
    return _classifier(x_out, W_cls1, b_cls1, gamma, beta, W_cls2, b_cls2)


# speedup vs baseline: 8.1146x; 1.0588x over previous
import functools

import jax
import jax.numpy as jnp
from jax import lax
from jax.experimental import pallas as pl
from jax.experimental.pallas import tpu as pltpu
from jax.experimental.pallas import tpu_sc as plsc

_BIG = 3.0e38


def _aug_body(r_ref, o_ref):
    o_ref[...] = jnp.sum(r_ref[...] * r_ref[...], axis=1)[None, :]


def _augment(r):
    nr = r.shape[0]
    return pl.pallas_call(
        _aug_body,
        in_specs=[pl.BlockSpec((nr, 3), lambda: (0, 0))],
        out_specs=pl.BlockSpec((1, nr), lambda: (0, 0)),
        out_shape=jax.ShapeDtypeStruct((1, nr), jnp.float32),
    )(r)


def _knn_body(k, with_w, nr, q_ref, r_ref, rn_ref, idx_ref, *maybe_w_ref):
    q = q_ref[...]
    r = r_ref[...]
    qn = jnp.sum(q * q, axis=1, keepdims=True)
    rn = rn_ref[...]
    qr = lax.dot_general(q, r, (((1,), (1,)), ((), ())),
                         preferred_element_type=jnp.float32)
    d = qn - 2.0 * qr + rn
    iota = lax.broadcasted_iota(jnp.int32, d.shape, 1).astype(jnp.float32)
    nrf = float(nr)
    cols = []
    vals = []
    for _ in range(k):
        m = jnp.min(d, axis=1, keepdims=True)
        cand = jnp.where(d == m, iota, nrf)
        col = jnp.min(cand, axis=1, keepdims=True)
        cols.append(col)
        vals.append(m)
        d = jnp.where(cand == col, _BIG, d)
    idx_ref[...] = jnp.concatenate(cols, axis=1).astype(jnp.int32)
    if with_w:
        d2 = jnp.maximum(jnp.concatenate(vals, axis=1), 0.0)
        w = 1.0 / (d2 + 1e-8)
        maybe_w_ref[0][...] = w / jnp.sum(w, axis=1, keepdims=True)


def _knn(q, r, rn, k, with_w, block_rows=128):
    nq = q.shape[0]
    nr = r.shape[0]
    bq = min(block_rows, nq)
    grid = (nq // bq,)
    out_shape = [jax.ShapeDtypeStruct((nq, k), jnp.int32)]
    out_specs = [pl.BlockSpec((bq, k), lambda i: (i, 0))]
    if with_w:
        out_shape.append(jax.ShapeDtypeStruct((nq, k), jnp.float32))
        out_specs.append(pl.BlockSpec((bq, k), lambda i: (i, 0)))
    res = pl.pallas_call(
        functools.partial(_knn_body, k, with_w, nr),
        grid=grid,
        in_specs=[pl.BlockSpec((bq, 3), lambda i: (i, 0)),
                  pl.BlockSpec((nr, 3), lambda i: (0, 0)),
                  pl.BlockSpec((1, nr), lambda i: (0, 0))],
        out_specs=out_specs,
        out_shape=out_shape,
    )(q, r, rn)
    return res if with_w else res[0]


def _dense_body(n_in, *refs):
    out_ref = refs[-1]
    acc = None
    for i in range(n_in):
        part = jnp.dot(refs[i][...], refs[n_in + i][...],
                       preferred_element_type=jnp.float32)
        acc = part if acc is None else acc + part
    out_ref[...] = jnp.maximum(acc, 0.0)


def _dense_relu(xs, ws, block_rows=512):
    m = xs[0].shape[0]
    bm = min(block_rows, m)
    n = ws[0].shape[1]
    in_specs = []
    for x in xs:
        kd = x.shape[1]
        in_specs.append(pl.BlockSpec((bm, kd), lambda i: (i, 0)))
    for w in ws:
        in_specs.append(pl.BlockSpec(w.shape, lambda i: (0, 0)))
    return pl.pallas_call(
        functools.partial(_dense_body, len(xs)),
        grid=(m // bm,),
        in_specs=in_specs,
        out_specs=pl.BlockSpec((bm, n), lambda i: (i, 0)),
        out_shape=jax.ShapeDtypeStruct((m, n), jnp.float32),
    )(*xs, *ws)


def _edge_body(a_ref, b_ref, w_ref, o_ref):
    o_ref[...] = jnp.maximum(
        jnp.dot(a_ref[...] - b_ref[...], w_ref[...],
                preferred_element_type=jnp.float32), 0.0)


def _edge_mm(a, b, w, block_rows=512):
    m, kd = a.shape
    n = w.shape[1]
    bm = min(block_rows, m)
    return pl.pallas_call(
        _edge_body,
        grid=(m // bm,),
        in_specs=[pl.BlockSpec((bm, kd), lambda i: (i, 0)),
                  pl.BlockSpec((bm, kd), lambda i: (i, 0)),
                  pl.BlockSpec(w.shape, lambda i: (0, 0))],
        out_specs=pl.BlockSpec((bm, n), lambda i: (i, 0)),
        out_shape=jax.ShapeDtypeStruct((m, n), jnp.float32),
    )(a, b, w)


def _cls_body(x_ref, w1_ref, b1_ref, g_ref, be_ref, w2_ref, b2_ref, o_ref):
    h = jnp.dot(x_ref[...], w1_ref[...],
                preferred_element_type=jnp.float32) + b1_ref[...]
    mu = jnp.mean(h, axis=0, keepdims=True)
    var = jnp.mean((h - mu) * (h - mu), axis=0, keepdims=True)
    h = g_ref[...] * (h - mu) / jnp.sqrt(var + 1e-5) + be_ref[...]
    h = jnp.maximum(h, 0.0)
    o_ref[...] = jnp.dot(h, w2_ref[...],
                         preferred_element_type=jnp.float32) + b2_ref[...]


def _classifier(x, w1, b1, g, be, w2, b2):
    m, d = x.shape
    n = w2.shape[1]
    full = lambda s: pl.BlockSpec(s, lambda: tuple(0 for _ in s))
    return pl.pallas_call(
        _cls_body,
        in_specs=[full((m, d)), full(w1.shape), full((1, d)), full((1, d)),
                  full((1, d)), full(w2.shape), full((1, n))],
        out_specs=full((m, n)),
        out_shape=jax.ShapeDtypeStruct((m, n), jnp.float32),
    )(x, w1, b1.reshape(1, -1), g.reshape(1, -1), be.reshape(1, -1),
      w2, b2.reshape(1, -1))


def _pad128(table):
    d = table.shape[1]
    dp = (d + 127) // 128 * 128
    if dp != d:
        table = jnp.pad(table, ((0, 0), (0, dp - d)))
    return table, d


def _sc_gather_reduce(table, idx, w):
    with_w = w is not None
    table, d_orig = _pad128(table)
    q_tot, kk = idx.shape
    _, d = table.shape
    info = plsc.get_sparse_core_info()
    nw = info.num_cores * info.num_subcores
    qpw = q_tot // nw
    b = max(1, min(128 // kk, qpw))
    nqb = qpw // b
    mesh = plsc.VectorSubcoreMesh(core_axis_name="c", subcore_axis_name="s")

    scratch = [pltpu.VMEM((qpw * kk,), jnp.int32)]
    if with_w:
        scratch.append(pltpu.VMEM((qpw * kk + 16,), jnp.float32))
    scratch += [
        pltpu.VMEM((b * kk, d), jnp.float32),
        pltpu.VMEM((b * kk, d), jnp.float32),
        pltpu.VMEM((b, d), jnp.float32),
        pltpu.VMEM((b, d), jnp.float32),
        pltpu.SemaphoreType.DMA,
        pltpu.SemaphoreType.DMA,
        pltpu.SemaphoreType.DMA,
        pltpu.SemaphoreType.DMA,
    ]

    @functools.partial(
        pl.kernel, mesh=mesh,
        out_type=jax.ShapeDtypeStruct((q_tot, d), jnp.float32),
        scratch_types=scratch,
    )
    def kern(*refs):
        if with_w:
            (table_hbm, idx_hbm, w_hbm, out_hbm, idx_v, w_v,
             rows0, rows1, out0, out1, sg0, sg1, ss0, ss1) = refs
        else:
            (table_hbm, idx_hbm, out_hbm, idx_v,
             rows0, rows1, out0, out1, sg0, sg1, ss0, ss1) = refs
        rows = (rows0, rows1)
        outs = (out0, out1)
        sgs = (sg0, sg1)
        sss = (ss0, ss1)
        wid = lax.axis_index("s") * info.num_cores + lax.axis_index("c")
        base = wid * qpw
        pltpu.sync_copy(idx_hbm.at[pl.ds(base * kk, qpw * kk)], idx_v)
        if with_w:
            pltpu.sync_copy(w_hbm.at[pl.ds(base * kk, qpw * kk)],
                            w_v.at[pl.ds(0, qpw * kk)])

        def gather(qb, s):
            return pltpu.make_async_copy(
                table_hbm.at[idx_v.at[pl.ds(qb * b * kk, b * kk)]],
                rows[s], sgs[s])

        def store(qb, s):
            return pltpu.make_async_copy(
                outs[s], out_hbm.at[pl.ds(base + qb * b, b)], sss[s])

        def compute(qb, s):
            rv, ov = rows[s], outs[s]

            def b_body(bi):
                if with_w:
                    wvec = w_v[pl.ds((qb * b + bi) * kk, 16)]
                    wjs = [wvec[j] for j in range(kk)]
                for dch in range(d // 16):
                    sl = pl.ds(dch * 16, 16)
                    if with_w:
                        acc = wjs[0] * rv[bi * kk, sl]
                        for j in range(1, kk):
                            acc = acc + wjs[j] * rv[bi * kk + j, sl]
                    else:
                        acc = rv[bi * kk, sl]
                        for j in range(1, kk):
                            acc = jnp.maximum(acc, rv[bi * kk + j, sl])
                    ov[bi, sl] = acc
            lax.fori_loop(0, b, lambda i, _: (b_body(i), 0)[1], 0)

        gather(0, 0).start()
        nqb2 = nqb - (nqb % 2)

        if nqb2 >= 2:
            @pl.loop(0, nqb2, step=2)
            def _(qb):
                gather(qb + 1, 1).start()
                gather(qb, 0).wait()

                @pl.when(qb >= 2)
                def _():
                    store(qb - 2, 0).wait()
                compute(qb, 0)
                store(qb, 0).start()

                @pl.when(qb + 2 < nqb)
                def _():
                    gather(qb + 2, 0).start()
                gather(qb + 1, 1).wait()

                @pl.when(qb >= 2)
                def _():
                    store(qb - 1, 1).wait()
                compute(qb + 1, 1)
                store(qb + 1, 1).start()

        if nqb % 2:
            last = nqb - 1
            gather(last, 0).wait()
            if nqb2 >= 2:
                store(nqb2 - 2, 0).wait()
            compute(last, 0)
            store(last, 0).start()
            store(last, 0).wait()
            if nqb2 >= 2:
                store(nqb2 - 1, 1).wait()
        else:
            store(nqb - 2, 0).wait()
            store(nqb - 1, 1).wait()

    if with_w:
        out = kern(table, idx.reshape(-1), w.reshape(-1))
    else:
        out = kern(table, idx.reshape(-1))
    return out[:, :d_orig] if d_orig != d else out


def _sc_poolmax(table, idx):
    return _sc_gather_reduce(table, idx, None)


def _sc_interp(table, idx, w):
    return _sc_gather_reduce(table, idx, w)


def kernel(p, x, W_enc0, W_enc1, W_enc2, W_dec1, W_dec0, W_fusion, W_edge,
           W_ee, W_cls1, b_cls1, gamma, beta, W_cls2, b_cls2, o):
    del o
    n0 = p.shape[0]
    d0_dim, d1_dim, d2_dim = W_enc0.shape[1], W_enc1.shape[1], W_enc2.shape[1]

    p1 = p[::4]
    p2 = p1[::4]
    pn, p1n, p2n = _augment(p), _augment(p1), _augment(p2)
    idx_e = _knn(p, p, pn, 16, with_w=False)
    idx1 = idx_e[::4]
    idx2 = _knn(p2, p1, p1n, 16, with_w=False)
    idx_m2, w_m2 = _knn(p, p2, p2n, 8, with_w=True)
    idx_u1, w_u1 = idx_m2[::4], w_m2[::4]
    idx_u0, w_u0 = _knn(p, p1, p1n, 8, with_w=True)

    x0 = _dense_relu([x], [W_enc0])
    x1 = _dense_relu([_sc_poolmax(x0, idx1)], [W_enc1])
    x2 = _dense_relu([_sc_poolmax(x1, idx2)], [W_enc2])

    up1 = _sc_interp(x2, idx_u1, w_u1)
    d1 = _dense_relu([up1, x1],
                     [W_dec1[:d2_dim], W_dec1[d2_dim:]])
    up0 = _sc_interp(d1, idx_u0, w_u0)
    d0 = _dense_relu([up0, x0],
                     [W_dec0[:d1_dim], W_dec0[d1_dim:]])

    ms1 = _sc_interp(x1, idx_u0, w_u0)
    ms2 = _sc_interp(x2, idx_m2, w_m2)
    ws = [W_fusion[:d0_dim], W_fusion[d0_dim:d0_dim + d1_dim],
          W_fusion[d0_dim + d1_dim:]]
    ms_new = _dense_relu([x0, ms1, ms2], ws)

    pooled = _sc_poolmax(ms_new, idx_e)
    ms_edge = _edge_mm(pooled, ms_new, W_edge)

    x_out = _dense_relu([d0, ms_edge],
                        [W_ee[:d0_dim], W_ee[d0_dim:]])
    del n0
    return _classifier(x_out, W_cls1, b_cls1, gamma, beta, W_cls2, b_cls2)


# speedup vs baseline: 8.5897x; 1.0585x over previous
import functools

import jax
import jax.numpy as jnp
from jax import lax
from jax.experimental import pallas as pl
from jax.experimental.pallas import tpu as pltpu
from jax.experimental.pallas import tpu_sc as plsc

_BIG = 3.0e38


def _aug_body(r_ref, o_ref):
    o_ref[...] = jnp.sum(r_ref[...] * r_ref[...], axis=1)[None, :]


def _augment(r):
    nr = r.shape[0]
    return pl.pallas_call(
        _aug_body,
        in_specs=[pl.BlockSpec((nr, 3), lambda: (0, 0))],
        out_specs=pl.BlockSpec((1, nr), lambda: (0, 0)),
        out_shape=jax.ShapeDtypeStruct((1, nr), jnp.float32),
    )(r)


def _knn_body(k, with_w, nr, q_ref, r_ref, rn_ref, idx_ref, *maybe_w_ref):
    q = q_ref[...]
    r = r_ref[...]
    qn = jnp.sum(q * q, axis=1, keepdims=True)
    rn = rn_ref[...]
    qr = lax.dot_general(q, r, (((1,), (1,)), ((), ())),
                         preferred_element_type=jnp.float32)
    d = qn - 2.0 * qr + rn
    iota = lax.broadcasted_iota(jnp.int32, d.shape, 1).astype(jnp.float32)
    nrf = float(nr)
    cols = []
    vals = []
    for _ in range(k):
        m = jnp.min(d, axis=1, keepdims=True)
        cand = jnp.where(d == m, iota, nrf)
        col = jnp.min(cand, axis=1, keepdims=True)
        cols.append(col)
        vals.append(m)
        d = jnp.where(cand == col, _BIG, d)
    idx_ref[...] = jnp.concatenate(cols, axis=1).astype(jnp.int32)
    if with_w:
        d2 = jnp.maximum(jnp.concatenate(vals, axis=1), 0.0)
        w = 1.0 / (d2 + 1e-8)
        maybe_w_ref[0][...] = w / jnp.sum(w, axis=1, keepdims=True)


def _knn(q, r, rn, k, with_w, block_rows=256):
    nq = q.shape[0]
    nr = r.shape[0]
    bq = min(block_rows, nq)
    grid = (nq // bq,)
    out_shape = [jax.ShapeDtypeStruct((nq, k), jnp.int32)]
    out_specs = [pl.BlockSpec((bq, k), lambda i: (i, 0))]
    if with_w:
        out_shape.append(jax.ShapeDtypeStruct((nq, k), jnp.float32))
        out_specs.append(pl.BlockSpec((bq, k), lambda i: (i, 0)))
    res = pl.pallas_call(
        functools.partial(_knn_body, k, with_w, nr),
        grid=grid,
        in_specs=[pl.BlockSpec((bq, 3), lambda i: (i, 0)),
                  pl.BlockSpec((nr, 3), lambda i: (0, 0)),
                  pl.BlockSpec((1, nr), lambda i: (0, 0))],
        out_specs=out_specs,
        out_shape=out_shape,
    )(q, r, rn)
    return res if with_w else res[0]


def _dense_body(n_in, *refs):
    out_ref = refs[-1]
    acc = None
    for i in range(n_in):
        part = jnp.dot(refs[i][...], refs[n_in + i][...],
                       preferred_element_type=jnp.float32)
        acc = part if acc is None else acc + part
    out_ref[...] = jnp.maximum(acc, 0.0)


def _dense_relu(xs, ws, block_rows=512):
    m = xs[0].shape[0]
    bm = min(block_rows, m)
    n = ws[0].shape[1]
    in_specs = []
    for x in xs:
        kd = x.shape[1]
        in_specs.append(pl.BlockSpec((bm, kd), lambda i: (i, 0)))
    for w in ws:
        in_specs.append(pl.BlockSpec(w.shape, lambda i: (0, 0)))
    return pl.pallas_call(
        functools.partial(_dense_body, len(xs)),
        grid=(m // bm,),
        in_specs=in_specs,
        out_specs=pl.BlockSpec((bm, n), lambda i: (i, 0)),
        out_shape=jax.ShapeDtypeStruct((m, n), jnp.float32),
    )(*xs, *ws)


def _edge_body(a_ref, b_ref, w_ref, o_ref):
    o_ref[...] = jnp.maximum(
        jnp.dot(a_ref[...] - b_ref[...], w_ref[...],
                preferred_element_type=jnp.float32), 0.0)


def _edge_mm(a, b, w, block_rows=512):
    m, kd = a.shape
    n = w.shape[1]
    bm = min(block_rows, m)
    return pl.pallas_call(
        _edge_body,
        grid=(m // bm,),
        in_specs=[pl.BlockSpec((bm, kd), lambda i: (i, 0)),
                  pl.BlockSpec((bm, kd), lambda i: (i, 0)),
                  pl.BlockSpec(w.shape, lambda i: (0, 0))],
        out_specs=pl.BlockSpec((bm, n), lambda i: (i, 0)),
        out_shape=jax.ShapeDtypeStruct((m, n), jnp.float32),
    )(a, b, w)


def _cls_body(x_ref, w1_ref, b1_ref, g_ref, be_ref, w2_ref, b2_ref, o_ref):
    h = jnp.dot(x_ref[...], w1_ref[...],
                preferred_element_type=jnp.float32) + b1_ref[...]
    mu = jnp.mean(h, axis=0, keepdims=True)
    var = jnp.mean((h - mu) * (h - mu), axis=0, keepdims=True)
    h = g_ref[...] * (h - mu) / jnp.sqrt(var + 1e-5) + be_ref[...]
    h = jnp.maximum(h, 0.0)
    o_ref[...] = jnp.dot(h, w2_ref[...],
                         preferred_element_type=jnp.float32) + b2_ref[...]


def _classifier(x, w1, b1, g, be, w2, b2):
    m, d = x.shape
    n = w2.shape[1]
    full = lambda s: pl.BlockSpec(s, lambda: tuple(0 for _ in s))
    return pl.pallas_call(
        _cls_body,
        in_specs=[full((m, d)), full(w1.shape), full((1, d)), full((1, d)),
                  full((1, d)), full(w2.shape), full((1, n))],
        out_specs=full((m, n)),
        out_shape=jax.ShapeDtypeStruct((m, n), jnp.float32),
    )(x, w1, b1.reshape(1, -1), g.reshape(1, -1), be.reshape(1, -1),
      w2, b2.reshape(1, -1))


def _pad128(table):
    d = table.shape[1]
    dp = (d + 127) // 128 * 128
    if dp != d:
        table = jnp.pad(table, ((0, 0), (0, dp - d)))
    return table, d


def _sc_gather_reduce(table, idx, w):
    with_w = w is not None
    table, d_orig = _pad128(table)
    q_tot, kk = idx.shape
    _, d = table.shape
    info = plsc.get_sparse_core_info()
    nw = info.num_cores * info.num_subcores
    qpw = q_tot // nw
    b = max(1, min(128 // kk, qpw))
    nqb = qpw // b
    mesh = plsc.VectorSubcoreMesh(core_axis_name="c", subcore_axis_name="s")

    scratch = [pltpu.VMEM((qpw * kk,), jnp.int32)]
    if with_w:
        scratch.append(pltpu.VMEM((qpw * kk + 16,), jnp.float32))
    scratch += [
        pltpu.VMEM((b * kk, d), jnp.float32),
        pltpu.VMEM((b * kk, d), jnp.float32),
        pltpu.VMEM((b, d), jnp.float32),
        pltpu.VMEM((b, d), jnp.float32),
        pltpu.SemaphoreType.DMA,
        pltpu.SemaphoreType.DMA,
        pltpu.SemaphoreType.DMA,
        pltpu.SemaphoreType.DMA,
    ]

    @functools.partial(
        pl.kernel, mesh=mesh,
        out_type=jax.ShapeDtypeStruct((q_tot, d), jnp.float32),
        scratch_types=scratch,
    )
    def kern(*refs):
        if with_w:
            (table_hbm, idx_hbm, w_hbm, out_hbm, idx_v, w_v,
             rows0, rows1, out0, out1, sg0, sg1, ss0, ss1) = refs
        else:
            (table_hbm, idx_hbm, out_hbm, idx_v,
             rows0, rows1, out0, out1, sg0, sg1, ss0, ss1) = refs
        rows = (rows0, rows1)
        outs = (out0, out1)
        sgs = (sg0, sg1)
        sss = (ss0, ss1)
        wid = lax.axis_index("s") * info.num_cores + lax.axis_index("c")
        base = wid * qpw
        pltpu.sync_copy(idx_hbm.at[pl.ds(base * kk, qpw * kk)], idx_v)
        if with_w:
            pltpu.sync_copy(w_hbm.at[pl.ds(base * kk, qpw * kk)],
                            w_v.at[pl.ds(0, qpw * kk)])

        def gather(qb, s):
            return pltpu.make_async_copy(
                table_hbm.at[idx_v.at[pl.ds(qb * b * kk, b * kk)]],
                rows[s], sgs[s])

        def store(qb, s):
            return pltpu.make_async_copy(
                outs[s], out_hbm.at[pl.ds(base + qb * b, b)], sss[s])

        def compute(qb, s):
            rv, ov = rows[s], outs[s]

            def b_body(bi):
                if with_w:
                    wvec = w_v[pl.ds((qb * b + bi) * kk, 16)]
                    wjs = [wvec[j] for j in range(kk)]
                for dch in range(d // 16):
                    sl = pl.ds(dch * 16, 16)
                    if with_w:
                        acc = wjs[0] * rv[bi * kk, sl]
                        for j in range(1, kk):
                            acc = acc + wjs[j] * rv[bi * kk + j, sl]
                    else:
                        acc = rv[bi * kk, sl]
                        for j in range(1, kk):
                            acc = jnp.maximum(acc, rv[bi * kk + j, sl])
                    ov[bi, sl] = acc
            lax.fori_loop(0, b, lambda i, _: (b_body(i), 0)[1], 0)

        gather(0, 0).start()
        nqb2 = nqb - (nqb % 2)

        if nqb2 >= 2:
            @pl.loop(0, nqb2, step=2)
            def _(qb):
                gather(qb + 1, 1).start()
                gather(qb, 0).wait()

                @pl.when(qb >= 2)
                def _():
                    store(qb - 2, 0).wait()
                compute(qb, 0)
                store(qb, 0).start()

                @pl.when(qb + 2 < nqb)
                def _():
                    gather(qb + 2, 0).start()
                gather(qb + 1, 1).wait()

                @pl.when(qb >= 2)
                def _():
                    store(qb - 1, 1).wait()
                compute(qb + 1, 1)
                store(qb + 1, 1).start()

        if nqb % 2:
            last = nqb - 1
            gather(last, 0).wait()
            if nqb2 >= 2:
                store(nqb2 - 2, 0).wait()
            compute(last, 0)
            store(last, 0).start()
            store(last, 0).wait()
            if nqb2 >= 2:
                store(nqb2 - 1, 1).wait()
        else:
            store(nqb - 2, 0).wait()
            store(nqb - 1, 1).wait()

    if with_w:
        out = kern(table, idx.reshape(-1), w.reshape(-1))
    else:
        out = kern(table, idx.reshape(-1))
    return out[:, :d_orig] if d_orig != d else out


def _sc_poolmax(table, idx):
    return _sc_gather_reduce(table, idx, None)


def _sc_interp(table, idx, w):
    return _sc_gather_reduce(table, idx, w)


def kernel(p, x, W_enc0, W_enc1, W_enc2, W_dec1, W_dec0, W_fusion, W_edge,
           W_ee, W_cls1, b_cls1, gamma, beta, W_cls2, b_cls2, o):
    del o
    n0 = p.shape[0]
    d0_dim, d1_dim, d2_dim = W_enc0.shape[1], W_enc1.shape[1], W_enc2.shape[1]

    p1 = p[::4]
    p2 = p1[::4]
    pn, p1n, p2n = _augment(p), _augment(p1), _augment(p2)
    idx_e = _knn(p, p, pn, 16, with_w=False)
    idx1 = idx_e[::4]
    idx2 = _knn(p2, p1, p1n, 16, with_w=False)
    idx_m2, w_m2 = _knn(p, p2, p2n, 8, with_w=True)
    idx_u1, w_u1 = idx_m2[::4], w_m2[::4]
    idx_u0, w_u0 = _knn(p, p1, p1n, 8, with_w=True)

    x0 = _dense_relu([x], [W_enc0])
    x1 = _dense_relu([_sc_poolmax(x0, idx1)], [W_enc1])
    x2 = _dense_relu([_sc_poolmax(x1, idx2)], [W_enc2])

    up1 = _sc_interp(x2, idx_u1, w_u1)
    d1 = _dense_relu([up1, x1],
                     [W_dec1[:d2_dim], W_dec1[d2_dim:]])
    up0 = _sc_interp(d1, idx_u0, w_u0)
    d0 = _dense_relu([up0, x0],
                     [W_dec0[:d1_dim], W_dec0[d1_dim:]])

    ms1 = _sc_interp(x1, idx_u0, w_u0)
    ms2 = _sc_interp(x2, idx_m2, w_m2)
    ws = [W_fusion[:d0_dim], W_fusion[d0_dim:d0_dim + d1_dim],
          W_fusion[d0_dim + d1_dim:]]
    ms_new = _dense_relu([x0, ms1, ms2], ws)

    pooled = _sc_poolmax(ms_new, idx_e)
    ms_edge = _edge_mm(pooled, ms_new, W_edge)

    x_out = _dense_relu([d0, ms_edge],
                        [W_ee[:d0_dim], W_ee[d0_dim:]])
    del n0
    return _classifier(x_out, W_cls1, b_cls1, gamma, beta, W_cls2, b_cls2)
